# Initial kernel scaffold; baseline (speedup 1.0000x reference)
import jax
import jax.numpy as jnp
from jax.experimental import pallas as pl


def kernel(x, edge_index, edge_attr, eW1, eb1, eW2, eb2, eW3, eb3, e_gamma, e_beta, nW1, nb1, nW2, nb2, nW3, nb3, n_gamma, n_beta):
    raise NotImplementedError("write your pallas kernel here")


# speedup vs baseline: 3.7104x; 3.7104x over previous
import functools

import jax
import jax.numpy as jnp
from jax import lax
from jax.experimental import pallas as pl
from jax.experimental.pallas import tpu as pltpu
from jax.experimental.pallas import tpu_sc as plsc

N_NODES = 10000
N_EDGES = 320000
D = 128
F = 16

NC, NS = 2, 16
NW = NC * NS
EPW = N_EDGES // NW
CH = 80
RND = 2000
GPR = RND // CH
NRND = EPW // RND
NPS = N_NODES // NS

def _sc_gather_body(pi_hbm, pj_hbm, ii_hbm, ij_hbm, gi_hbm, gj_hbm,
                    ii_v, ij_v, rows_v, sem):
    cid = lax.axis_index("c")
    sid = lax.axis_index("s")
    wid = sid * NC + cid
    base = wid * EPW
    pltpu.sync_copy(ii_hbm.at[pl.ds(base, EPW)], ii_v)
    pltpu.sync_copy(ij_hbm.at[pl.ds(base, EPW)], ij_v)

    def make_round(idx_v, tab_hbm, out_hbm):
        def round_body(o, carry):
            cps = []
            for k in range(GPR):
                s = o * RND + k * CH
                cps.append(pltpu.async_copy(
                    tab_hbm.at[idx_v.at[pl.ds(s, CH)]],
                    rows_v.at[pl.ds(k * CH, CH)], sem))
            for cp in cps:
                cp.wait()
            pltpu.sync_copy(rows_v, out_hbm.at[pl.ds(base + o * RND, RND)])
            return carry
        return round_body

    lax.fori_loop(0, NRND, make_round(ii_v, pi_hbm, gi_hbm), 0)
    lax.fori_loop(0, NRND, make_round(ij_v, pj_hbm, gj_hbm), 0)


@functools.cache
def _gather_call():
    return pl.kernel(
        _sc_gather_body,
        mesh=plsc.VectorSubcoreMesh(core_axis_name="c", subcore_axis_name="s"),
        out_type=[jax.ShapeDtypeStruct((N_EDGES, F), jnp.float32),
                  jax.ShapeDtypeStruct((N_EDGES, F), jnp.float32)],
        scratch_types=[pltpu.VMEM((EPW,), jnp.int32),
                       pltpu.VMEM((EPW,), jnp.int32),
                       pltpu.VMEM((RND, F), jnp.float32),
                       pltpu.SemaphoreType.DMA],
        compiler_params=pltpu.CompilerParams(use_tc_tiling_on_sc=False),
    )


def _sc_scatter_body(rows_hbm, idx2_hbm, zero_hbm, out_hbm,
                     idx_v, rows_v, acc_sh):
    cid = lax.axis_index("c")
    sid = lax.axis_index("s")
    wid = sid * NC + cid
    zslice = pl.ds(sid * NPS, NPS)
    pltpu.sync_copy(zero_hbm.at[zslice], acc_sh.at[zslice])
    plsc.subcore_barrier()

    rpw = EPW // CH
    pltpu.sync_copy(idx2_hbm.at[pl.ds(wid * rpw, rpw)], idx_v)

    def round_body(o, carry):
        pltpu.sync_copy(rows_hbm.at[pl.ds(wid * EPW + o * RND, RND)], rows_v)
        for k in range(GPR):
            pltpu.sync_copy(rows_v.at[pl.ds(k * CH, CH)],
                            acc_sh.at[idx_v.at[o * GPR + k]], add=True)
        return carry

    lax.fori_loop(0, NRND, round_body, 0)
    plsc.subcore_barrier()
    pltpu.sync_copy(acc_sh.at[zslice], out_hbm.at[cid, zslice])


@functools.cache
def _scatter_call():
    return pl.kernel(
        _sc_scatter_body,
        mesh=plsc.VectorSubcoreMesh(core_axis_name="c", subcore_axis_name="s"),
        out_type=jax.ShapeDtypeStruct((NC, N_NODES, F), jnp.float32),
        scratch_types=[pltpu.VMEM((EPW // CH, CH), jnp.int32),
                       pltpu.VMEM((RND, F), jnp.float32),
                       pltpu.VMEM_SHARED((N_NODES, F), jnp.float32)],
        compiler_params=pltpu.CompilerParams(use_tc_tiling_on_sc=False),
    )


def _prep_body(x_ref, wi_ref, wj_ref, pi_ref, pj_ref):
    xv = x_ref[...]
    pi_ref[...] = jnp.dot(xv, wi_ref[...], preferred_element_type=jnp.float32)
    pj_ref[...] = jnp.dot(xv, wj_ref[...], preferred_element_type=jnp.float32)


def _edge_body(gi_ref, gj_ref, ea_ref, w1_ref, b1_ref, w2_ref, b2_ref,
               w3_ref, b3_ref, g_ref, bt_ref, out_ref):
    ea = ea_ref[...]
    h = (gi_ref[...] + gj_ref[...]
         + jnp.dot(ea, w1_ref[...], preferred_element_type=jnp.float32)
         + b1_ref[...])
    h = h * lax.logistic(h)
    h = jnp.dot(h, w2_ref[...], preferred_element_type=jnp.float32) + b2_ref[...]
    h = h * lax.logistic(h)
    h = jnp.dot(h, w3_ref[...], preferred_element_type=jnp.float32) + b3_ref[...]
    mu = jnp.mean(h, axis=-1, keepdims=True)
    d = h - mu
    var = jnp.mean(d * d, axis=-1, keepdims=True)
    out_ref[...] = ea + d * lax.rsqrt(var + 1e-5) * g_ref[...] + bt_ref[...]


def _node_body(x_ref, a0_ref, a1_ref, w1x_ref, w1a_ref, b1_ref, w2_ref, b2_ref,
               w3_ref, b3_ref, g_ref, bt_ref, out_ref):
    xv = x_ref[...]
    a = a0_ref[...] + a1_ref[...]
    h = (jnp.dot(xv, w1x_ref[...], preferred_element_type=jnp.float32)
         + jnp.dot(a, w1a_ref[...], preferred_element_type=jnp.float32)
         + b1_ref[...])
    h = h * lax.logistic(h)
    h = jnp.dot(h, w2_ref[...], preferred_element_type=jnp.float32) + b2_ref[...]
    h = h * lax.logistic(h)
    h = jnp.dot(h, w3_ref[...], preferred_element_type=jnp.float32) + b3_ref[...]
    mu = jnp.mean(h, axis=-1, keepdims=True)
    d = h - mu
    var = jnp.mean(d * d, axis=-1, keepdims=True)
    out_ref[...] = xv + d * lax.rsqrt(var + 1e-5) * g_ref[...] + bt_ref[...]


_NB = 2000
_EB = 8000

_full = lambda shape: pl.BlockSpec(shape, lambda i: (0,) * len(shape))


def _prep_call(x, wi, wj):
    g = N_NODES // _NB
    return pl.pallas_call(
        _prep_body,
        grid=(g,),
        in_specs=[pl.BlockSpec((_NB, D), lambda i: (i, 0)),
                  _full((D, F)), _full((D, F))],
        out_specs=[pl.BlockSpec((_NB, F), lambda i: (i, 0)),
                   pl.BlockSpec((_NB, F), lambda i: (i, 0))],
        out_shape=[jax.ShapeDtypeStruct((N_NODES, F), jnp.float32),
                   jax.ShapeDtypeStruct((N_NODES, F), jnp.float32)],
    )(x, wi, wj)


def _edge_call(gi, gj, ea, w1, b1, w2, b2, w3, b3, gm, bt):
    g = N_EDGES // _EB
    eb = lambda: pl.BlockSpec((_EB, F), lambda i: (i, 0))
    return pl.pallas_call(
        _edge_body,
        grid=(g,),
        in_specs=[eb(), eb(), eb(),
                  _full((F, F)), _full((1, F)), _full((F, F)), _full((1, F)),
                  _full((F, F)), _full((1, F)), _full((1, F)), _full((1, F))],
        out_specs=eb(),
        out_shape=jax.ShapeDtypeStruct((N_EDGES, F), jnp.float32),
    )(gi, gj, ea, w1, b1, w2, b2, w3, b3, gm, bt)


def _node_call(x, a0, a1, w1x, w1a, b1, w2, b2, w3, b3, gm, bt):
    g = N_NODES // _NB
    return pl.pallas_call(
        _node_body,
        grid=(g,),
        in_specs=[pl.BlockSpec((_NB, D), lambda i: (i, 0)),
                  pl.BlockSpec((_NB, F), lambda i: (i, 0)),
                  pl.BlockSpec((_NB, F), lambda i: (i, 0)),
                  _full((D, D)), _full((F, D)), _full((1, D)),
                  _full((D, D)), _full((1, D)),
                  _full((D, D)), _full((1, D)),
                  _full((1, D)), _full((1, D))],
        out_specs=pl.BlockSpec((_NB, D), lambda i: (i, 0)),
        out_shape=jax.ShapeDtypeStruct((N_NODES, D), jnp.float32),
    )(x, a0, a1, w1x, w1a, b1, w2, b2, w3, b3, gm, bt)


def kernel(x, edge_index, edge_attr,
           eW1, eb1, eW2, eb2, eW3, eb3, e_gamma, e_beta,
           nW1, nb1, nW2, nb2, nW3, nb3, n_gamma, n_beta):
    f32 = jnp.float32
    idx_i = edge_index[0].astype(jnp.int32)
    idx_j = edge_index[1].astype(jnp.int32)

    pi, pj = _prep_call(x, eW1[:D], eW1[D:2 * D])
    gi, gj = _gather_call()(pi, pj, idx_i, idx_j)

    edge_new = _edge_call(
        gi, gj, edge_attr,
        eW1[2 * D:], eb1[None], eW2, eb2[None], eW3, eb3[None],
        e_gamma[None], e_beta[None])

    idx2 = idx_j.reshape(N_EDGES // CH, CH)
    partial = _scatter_call()(edge_new, idx2, jnp.zeros((N_NODES, F), f32))

    x_new = _node_call(
        x, partial[0], partial[1],
        nW1[:D], nW1[D:], nb1[None], nW2, nb2[None], nW3, nb3[None],
        n_gamma[None], n_beta[None])

    return (x_new, edge_new)


# speedup vs baseline: 7.4713x; 2.0136x over previous
import functools

import jax
import jax.numpy as jnp
from jax import lax
from jax.experimental import pallas as pl
from jax.experimental.pallas import tpu as pltpu
from jax.experimental.pallas import tpu_sc as plsc

N_NODES = 10000
N_EDGES = 320000
D = 128
F = 16

NC, NS = 2, 16
NW = NC * NS
EPW = N_EDGES // NW
CH = 80
RND = 2000
GPR = RND // CH
NRND = EPW // RND
NPS = N_NODES // NS

def _sc_gather_body(pi_hbm, pj_hbm, ii_hbm, ij_hbm, gi_hbm, gj_hbm,
                    ii_v, ij_v, rows_v, sem):
    cid = lax.axis_index("c")
    sid = lax.axis_index("s")
    wid = sid * NC + cid
    base = wid * EPW
    pltpu.sync_copy(ii_hbm.at[pl.ds(base, EPW)], ii_v)
    pltpu.sync_copy(ij_hbm.at[pl.ds(base, EPW)], ij_v)

    def make_round(idx_v, tab_hbm, out_hbm):
        def round_body(o, carry):
            cps = []
            for k in range(GPR):
                s = o * RND + k * CH
                cps.append(pltpu.async_copy(
                    tab_hbm.at[idx_v.at[pl.ds(s, CH)]],
                    rows_v.at[pl.ds(k * CH, CH)], sem))
            for cp in cps:
                cp.wait()
            pltpu.sync_copy(rows_v, out_hbm.at[pl.ds(base + o * RND, RND)])
            return carry
        return round_body

    lax.fori_loop(0, NRND, make_round(ii_v, pi_hbm, gi_hbm), 0)
    lax.fori_loop(0, NRND, make_round(ij_v, pj_hbm, gj_hbm), 0)


@functools.cache
def _gather_call():
    return pl.kernel(
        _sc_gather_body,
        mesh=plsc.VectorSubcoreMesh(core_axis_name="c", subcore_axis_name="s"),
        out_type=[jax.ShapeDtypeStruct((N_EDGES, F), jnp.float32),
                  jax.ShapeDtypeStruct((N_EDGES, F), jnp.float32)],
        scratch_types=[pltpu.VMEM((EPW,), jnp.int32),
                       pltpu.VMEM((EPW,), jnp.int32),
                       pltpu.VMEM((RND, F), jnp.float32),
                       pltpu.SemaphoreType.DMA],
        compiler_params=pltpu.CompilerParams(use_tc_tiling_on_sc=False),
    )


def _sc_scatter_body(rows_hbm, idx2_hbm, zero_hbm, out_hbm,
                     idx_v, rows_v, acc_sh):
    cid = lax.axis_index("c")
    sid = lax.axis_index("s")
    wid = sid * NC + cid
    zslice = pl.ds(sid * NPS, NPS)
    pltpu.sync_copy(zero_hbm.at[zslice], acc_sh.at[zslice])
    plsc.subcore_barrier()

    rpw = EPW // CH
    pltpu.sync_copy(idx2_hbm.at[pl.ds(wid * rpw, rpw)], idx_v)

    def round_body(o, carry):
        pltpu.sync_copy(rows_hbm.at[pl.ds(wid * EPW + o * RND, RND)], rows_v)
        for k in range(GPR):
            pltpu.sync_copy(rows_v.at[pl.ds(k * CH, CH)],
                            acc_sh.at[idx_v.at[o * GPR + k]], add=True)
        return carry

    lax.fori_loop(0, NRND, round_body, 0)
    plsc.subcore_barrier()
    pltpu.sync_copy(acc_sh.at[zslice], out_hbm.at[cid, zslice])


@functools.cache
def _scatter_call():
    return pl.kernel(
        _sc_scatter_body,
        mesh=plsc.VectorSubcoreMesh(core_axis_name="c", subcore_axis_name="s"),
        out_type=jax.ShapeDtypeStruct((NC, N_NODES, F), jnp.float32),
        scratch_types=[pltpu.VMEM((EPW // CH, CH), jnp.int32),
                       pltpu.VMEM((RND, F), jnp.float32),
                       pltpu.VMEM_SHARED((N_NODES, F), jnp.float32)],
        compiler_params=pltpu.CompilerParams(use_tc_tiling_on_sc=False),
    )


def _prep_body(x_ref, wi_ref, wj_ref, pi_ref, pj_ref):
    xv = x_ref[...]
    pi_ref[...] = jnp.dot(xv, wi_ref[...], preferred_element_type=jnp.float32)
    pj_ref[...] = jnp.dot(xv, wj_ref[...], preferred_element_type=jnp.float32)


def _edge_body(gi_ref, gj_ref, ea_ref, w1_ref, b1_ref, w2_ref, b2_ref,
               w3_ref, b3_ref, m_ref, g_ref, bt_ref, out_ref):
    ea = ea_ref[...]
    h = (gi_ref[...] + gj_ref[...]
         + jnp.dot(ea, w1_ref[...], preferred_element_type=jnp.float32)
         + b1_ref[...])
    h = h * lax.logistic(h)
    h = jnp.dot(h, w2_ref[...], preferred_element_type=jnp.float32) + b2_ref[...]
    h = h * lax.logistic(h)
    h = jnp.dot(h, w3_ref[...], preferred_element_type=jnp.float32) + b3_ref[...]
    m = m_ref[...]
    mu = jnp.dot(h, m, preferred_element_type=jnp.float32)
    d = h - mu
    var = jnp.dot(d * d, m, preferred_element_type=jnp.float32)
    out_ref[...] = ea + d * lax.rsqrt(var + 1e-5) * g_ref[...] + bt_ref[...]


def _node_body(x_ref, a0_ref, a1_ref, w1x_ref, w1a_ref, b1_ref, w2_ref, b2_ref,
               w3_ref, b3_ref, g_ref, bt_ref, out_ref):
    xv = x_ref[...]
    a = a0_ref[...] + a1_ref[...]
    h = (jnp.dot(xv, w1x_ref[...], preferred_element_type=jnp.float32)
         + jnp.dot(a, w1a_ref[...], preferred_element_type=jnp.float32)
         + b1_ref[...])
    h = h * lax.logistic(h)
    h = jnp.dot(h, w2_ref[...], preferred_element_type=jnp.float32) + b2_ref[...]
    h = h * lax.logistic(h)
    h = jnp.dot(h, w3_ref[...], preferred_element_type=jnp.float32) + b3_ref[...]
    mu = jnp.mean(h, axis=-1, keepdims=True)
    d = h - mu
    var = jnp.mean(d * d, axis=-1, keepdims=True)
    out_ref[...] = xv + d * lax.rsqrt(var + 1e-5) * g_ref[...] + bt_ref[...]


FOLD = D // F
EF_ROWS = N_EDGES // FOLD

_NB = 2000
_EB = 2000

_full = lambda shape: pl.BlockSpec(shape, lambda i: (0,) * len(shape))


def _prep_call(x, wi, wj):
    g = N_NODES // _NB
    return pl.pallas_call(
        _prep_body,
        grid=(g,),
        in_specs=[pl.BlockSpec((_NB, D), lambda i: (i, 0)),
                  _full((D, F)), _full((D, F))],
        out_specs=[pl.BlockSpec((_NB, F), lambda i: (i, 0)),
                   pl.BlockSpec((_NB, F), lambda i: (i, 0))],
        out_shape=[jax.ShapeDtypeStruct((N_NODES, F), jnp.float32),
                   jax.ShapeDtypeStruct((N_NODES, F), jnp.float32)],
    )(x, wi, wj)


def _edge_call(gi, gj, ea, w1, b1, w2, b2, w3, b3, m, gm, bt):
    g = EF_ROWS // _EB
    eb = lambda: pl.BlockSpec((_EB, D), lambda i: (i, 0))
    return pl.pallas_call(
        _edge_body,
        grid=(g,),
        in_specs=[eb(), eb(), eb(),
                  _full((D, D)), _full((1, D)), _full((D, D)), _full((1, D)),
                  _full((D, D)), _full((1, D)), _full((D, D)),
                  _full((1, D)), _full((1, D))],
        out_specs=eb(),
        out_shape=jax.ShapeDtypeStruct((EF_ROWS, D), jnp.float32),
    )(gi, gj, ea, w1, b1, w2, b2, w3, b3, m, gm, bt)


def _node_call(x, a0, a1, w1x, w1a, b1, w2, b2, w3, b3, gm, bt):
    g = N_NODES // _NB
    return pl.pallas_call(
        _node_body,
        grid=(g,),
        in_specs=[pl.BlockSpec((_NB, D), lambda i: (i, 0)),
                  pl.BlockSpec((_NB, F), lambda i: (i, 0)),
                  pl.BlockSpec((_NB, F), lambda i: (i, 0)),
                  _full((D, D)), _full((F, D)), _full((1, D)),
                  _full((D, D)), _full((1, D)),
                  _full((D, D)), _full((1, D)),
                  _full((1, D)), _full((1, D))],
        out_specs=pl.BlockSpec((_NB, D), lambda i: (i, 0)),
        out_shape=jax.ShapeDtypeStruct((N_NODES, D), jnp.float32),
    )(x, a0, a1, w1x, w1a, b1, w2, b2, w3, b3, gm, bt)


def kernel(x, edge_index, edge_attr,
           eW1, eb1, eW2, eb2, eW3, eb3, e_gamma, e_beta,
           nW1, nb1, nW2, nb2, nW3, nb3, n_gamma, n_beta):
    f32 = jnp.float32
    idx_i = edge_index[0].astype(jnp.int32)
    idx_j = edge_index[1].astype(jnp.int32)

    pi, pj = _prep_call(x, eW1[:D], eW1[D:2 * D])
    gi, gj = _gather_call()(pi, pj, idx_i, idx_j)

    eye8 = jnp.eye(FOLD, dtype=f32)
    w1d = jnp.kron(eye8, eW1[2 * D:])
    w2d = jnp.kron(eye8, eW2)
    w3d = jnp.kron(eye8, eW3)
    mavg = jnp.kron(eye8, jnp.full((F, F), 1.0 / F, f32))
    ef = _edge_call(
        gi.reshape(EF_ROWS, D), gj.reshape(EF_ROWS, D),
        edge_attr.reshape(EF_ROWS, D),
        w1d, jnp.tile(eb1, FOLD)[None], w2d, jnp.tile(eb2, FOLD)[None],
        w3d, jnp.tile(eb3, FOLD)[None], mavg,
        jnp.tile(e_gamma, FOLD)[None], jnp.tile(e_beta, FOLD)[None])
    edge_new = ef.reshape(N_EDGES, F)

    idx2 = idx_j.reshape(N_EDGES // CH, CH)
    partial = _scatter_call()(edge_new, idx2, jnp.zeros((N_NODES, F), f32))

    x_new = _node_call(
        x, partial[0], partial[1],
        nW1[:D], nW1[D:], nb1[None], nW2, nb2[None], nW3, nb3[None],
        n_gamma[None], n_beta[None])

    return (x_new, edge_new)
